# Initial kernel scaffold; baseline (speedup 1.0000x reference)
import jax
import jax.numpy as jnp
from jax.experimental import pallas as pl


def kernel(x, params):
    raise NotImplementedError("write your pallas kernel here")


# speedup vs baseline: 1.1642x; 1.1642x over previous
import functools

import jax
import jax.numpy as jnp
from jax.experimental import pallas as pl
from jax.experimental.pallas import tpu as pltpu

_C = 4

_PH2 = ((1, -1), (0, 0), (1, 0), (0, 1))
_PH16 = (
    (((3, -1), (0, 0), (1, 0), (2, 0))),
    (((1, 0), (2, 0), (3, 0), (0, 1))),
)
_TDEC = (((1, 0), (3, -1)), ((0, 1), (2, 0)))
_TQ = (
    ((0, 1, 0), (1, 3, -1)),
    ((0, 2, 0), (1, 0, 0)),
    ((0, 3, 0), (1, 1, 0)),
    ((0, 0, 1), (1, 2, 0)),
)


def _bf(a):
    return a.astype(jnp.bfloat16).astype(jnp.float32)


def _shift2(a, sy, sx):
    if sy == -1:
        a = jnp.concatenate([jnp.zeros_like(a[:, :1, :]), a[:, :-1, :]], axis=1)
    elif sy == 1:
        a = jnp.concatenate([a[:, 1:, :], jnp.zeros_like(a[:, :1, :])], axis=1)
    if sx == -1:
        a = jnp.concatenate([jnp.zeros_like(a[:, :, :1]), a[:, :, :-1]], axis=2)
    elif sx == 1:
        a = jnp.concatenate([a[:, :, 1:], jnp.zeros_like(a[:, :, :1])], axis=2)
    return a


def _ka_body(x2_ref, w1_ref, b1_ref, st_ref):
    i = pl.program_id(0)
    xb = {}
    for q in range(2):
        for r in range(2):
            xb[(q, r)] = _bf(x2_ref[q, r])
    xs = {}
    for dy in range(4):
        qy, sy = _PH2[dy]
        for dx in range(4):
            qx, sx = _PH2[dx]
            xs[(dy, dx)] = _shift2(xb[(qy, qx)], sy, sx)
    sums, sqs = [], []
    for o in range(16):
        acc = None
        for dy in range(4):
            for dx in range(4):
                t = _bf(w1_ref[o, dy, dx]) * xs[(dy, dx)]
                acc = t if acc is None else acc + t
        h = acc + b1_ref[o]
        sums.append(jnp.sum(h))
        sqs.append(jnp.sum(h * h))
    st = jnp.stack([jnp.stack(sums), jnp.stack(sqs)])

    @pl.when(i == 0)
    def _():
        st_ref[...] = jnp.zeros_like(st_ref)

    st_ref[...] += st


def _kb_body(x16_ref, w1_ref, s1_ref, b1_ref, w2_ref, b2_ref, h2_ref, st_ref):
    i = pl.program_id(0)
    xb = {}
    for py in range(4):
        for px in range(4):
            xb[(py, px)] = _bf(x16_ref[py, px])
    hn = {}
    for a in range(2):
        for b in range(2):
            xs = {}
            for dy in range(4):
                py, sy = _PH16[a][dy]
                for dx in range(4):
                    px, sx = _PH16[b][dx]
                    xs[(dy, dx)] = _shift2(xb[(py, px)], sy, sx)
            for o in range(16):
                acc = None
                for dy in range(4):
                    for dx in range(4):
                        t = _bf(w1_ref[o, dy, dx]) * s1_ref[o] * xs[(dy, dx)]
                        acc = t if acc is None else acc + t
                hn[(a, b, o)] = _bf(jnp.maximum(acc + b1_ref[o], 0.0))
    acc2 = [None] * 4
    for dy in range(4):
        a, sy = _PH2[dy]
        for dx in range(4):
            b, sx = _PH2[dx]
            for c in range(16):
                hs = _shift2(hn[(a, b, c)], sy, sx)
                for o in range(4):
                    t = _bf(w2_ref[o, c, dy, dx]) * hs
                    acc2[o] = t if acc2[o] is None else acc2[o] + t
    outs = [acc2[o] + b2_ref[o] for o in range(4)]
    h2_ref[...] = jnp.stack(outs, axis=1)
    sums = jnp.stack([jnp.sum(v) for v in outs])
    sqs = jnp.stack([jnp.sum(v * v) for v in outs])
    st = jnp.stack([sums, sqs])

    @pl.when(i == 0)
    def _():
        st_ref[...] = jnp.zeros_like(st_ref)

    st_ref[...] += st


def _kc_body(h2_ref, sc_ref, z_ref, loss_ref):
    i = pl.program_id(0)
    r = [
        _bf(jnp.maximum(h2_ref[:, c] * sc_ref[c] + sc_ref[4 + c], 0.0))
        for c in range(4)
    ]
    q0 = r[0] * _bf(sc_ref[8]) + r[1] * _bf(sc_ref[9]) + r[2] * _bf(sc_ref[10]) \
        + r[3] * _bf(sc_ref[11]) + sc_ref[16]
    q1 = r[0] * _bf(sc_ref[12]) + r[1] * _bf(sc_ref[13]) + r[2] * _bf(sc_ref[14]) \
        + r[3] * _bf(sc_ref[15]) + sc_ref[17]
    cb = [(sc_ref[18 + 2 * k], sc_ref[19 + 2 * k]) for k in range(3)]
    d = [(q0 - cb[k][0]) ** 2 + (q1 - cb[k][1]) ** 2 for k in range(3)]
    m1 = d[1] < d[0]
    bd = jnp.where(m1, d[1], d[0])
    e0 = jnp.where(m1, cb[1][0], cb[0][0])
    e1 = jnp.where(m1, cb[1][1], cb[0][1])
    m2 = d[2] < bd
    e0 = jnp.where(m2, cb[2][0], e0)
    e1 = jnp.where(m2, cb[2][1], e1)
    lsum = jnp.sum((e0 - q0) ** 2 + (e1 - q1) ** 2)
    eb0, eb1 = _bf(e0), _bf(e1)
    zs = [
        eb0 * _bf(sc_ref[24 + 2 * j]) + eb1 * _bf(sc_ref[25 + 2 * j])
        + sc_ref[32 + j]
        for j in range(4)
    ]
    z_ref[...] = jnp.stack(zs, axis=1)

    @pl.when(i == 0)
    def _():
        loss_ref[0] = 0.0

    loss_ref[0] += lsum


def _kd_body(z_ref, w_ref, b_ref, st_ref):
    i = pl.program_id(0)
    zc = [_bf(z_ref[:, c]) for c in range(4)]
    sums = [None] * 16
    sqs = [None] * 16
    for a in range(2):
        for b in range(2):
            sh = {}
            for ky, sy in _TDEC[a]:
                for kx, sx in _TDEC[b]:
                    for c in range(4):
                        sh[(ky, kx, c)] = _shift2(zc[c], sy, sx)
            for o in range(16):
                acc = None
                for ky, _ in _TDEC[a]:
                    for kx, _ in _TDEC[b]:
                        for c in range(4):
                            t = _bf(w_ref[c, o, ky, kx]) * sh[(ky, kx, c)]
                            acc = t if acc is None else acc + t
                dv = acc + b_ref[o]
                s = jnp.sum(dv)
                s2 = jnp.sum(dv * dv)
                sums[o] = s if sums[o] is None else sums[o] + s
                sqs[o] = s2 if sqs[o] is None else sqs[o] + s2
    st = jnp.stack([jnp.stack(sums), jnp.stack(sqs)])

    @pl.when(i == 0)
    def _():
        st_ref[...] = jnp.zeros_like(st_ref)

    st_ref[...] += st


def _ke_body(z_ref, w1_ref, s3_ref, b1_ref, w2_ref, b2_ref, out_ref):
    zc = [_bf(z_ref[:, c]) for c in range(4)]
    d1n = {}
    for a in range(2):
        for b in range(2):
            sh = {}
            for ky, sy in _TDEC[a]:
                for kx, sx in _TDEC[b]:
                    for c in range(4):
                        sh[(ky, kx, c)] = _shift2(zc[c], sy, sx)
            for o in range(16):
                acc = None
                for ky, _ in _TDEC[a]:
                    for kx, _ in _TDEC[b]:
                        for c in range(4):
                            t = _bf(w1_ref[c, o, ky, kx]) * s3_ref[o] \
                                * sh[(ky, kx, c)]
                            acc = t if acc is None else acc + t
                d1n[(a, b, o)] = _bf(jnp.maximum(acc + b1_ref[o], 0.0))
    cache = {}

    def getsh(a, b, sy, sx, c):
        k = (a, b, sy, sx, c)
        if k not in cache:
            cache[k] = _shift2(d1n[(a, b, c)], sy, sx)
        return cache[k]

    for ry in range(4):
        for rx in range(4):
            acc = None
            for ay, ky, sy in _TQ[ry]:
                for ax, kx, sx in _TQ[rx]:
                    for c in range(16):
                        t = _bf(w2_ref[c, ky, kx]) * getsh(ay, ax, sy, sx, c)
                        acc = t if acc is None else acc + t
            out_ref[ry, rx] = jnp.tanh(acc + b2_ref[0])


def _smem(shape):
    return pl.BlockSpec(memory_space=pltpu.SMEM)


def _pipeline(x, params):
    f32 = jnp.float32
    p = params
    B = x.shape[0]
    G = B // _C
    seq = pltpu.CompilerParams(dimension_semantics=("arbitrary",))

    x2 = x.reshape(B, 112, 2, 112, 2).transpose(2, 4, 0, 1, 3)
    x16 = x.reshape(B, 56, 4, 56, 4).transpose(2, 4, 0, 1, 3)

    st1 = pl.pallas_call(
        _ka_body,
        grid=(G,),
        in_specs=[
            pl.BlockSpec((2, 2, _C, 112, 112), lambda i: (0, 0, i, 0, 0)),
            _smem(None), _smem(None),
        ],
        out_specs=pl.BlockSpec((2, 16), lambda i: (0, 0)),
        out_shape=jax.ShapeDtypeStruct((2, 16), f32),
        compiler_params=seq,
    )(x2, p['enc_w1'][:, 0], p['enc_b1'])

    n1 = jnp.float32(B * 112 * 112)
    mu1 = st1[0] / n1
    var1 = st1[1] / n1 - mu1 * mu1
    s1 = p['bn1_g'] / jnp.sqrt(var1 + 1e-5)
    t1 = (p['enc_b1'] - mu1) * s1 + p['bn1_b']

    h2, st2 = pl.pallas_call(
        _kb_body,
        grid=(G,),
        in_specs=[
            pl.BlockSpec((4, 4, _C, 56, 56), lambda i: (0, 0, i, 0, 0)),
            _smem(None), _smem(None), _smem(None), _smem(None), _smem(None),
        ],
        out_specs=[
            pl.BlockSpec((_C, 4, 56, 56), lambda i: (i, 0, 0, 0)),
            pl.BlockSpec((2, 4), lambda i: (0, 0)),
        ],
        out_shape=[
            jax.ShapeDtypeStruct((B, 4, 56, 56), f32),
            jax.ShapeDtypeStruct((2, 4), f32),
        ],
        compiler_params=seq,
    )(x16, p['enc_w1'][:, 0], s1, t1, p['enc_w2'], p['enc_b2'])

    n2 = jnp.float32(B * 56 * 56)
    mu2 = st2[0] / n2
    var2 = st2[1] / n2 - mu2 * mu2
    s2 = p['bn2_g'] / jnp.sqrt(var2 + 1e-5)
    t2 = p['bn2_b'] - mu2 * s2

    sc = jnp.concatenate([
        s2, t2,
        p['pre_w'][:, :, 0, 0].reshape(-1), p['pre_b'],
        p['codebook'].reshape(-1),
        p['post_w'][:, :, 0, 0].reshape(-1), p['post_b'],
    ])
    z, lsum = pl.pallas_call(
        _kc_body,
        grid=(G,),
        in_specs=[
            pl.BlockSpec((_C, 4, 56, 56), lambda i: (i, 0, 0, 0)),
            _smem(None),
        ],
        out_specs=[
            pl.BlockSpec((_C, 4, 56, 56), lambda i: (i, 0, 0, 0)),
            pl.BlockSpec(memory_space=pltpu.SMEM),
        ],
        out_shape=[
            jax.ShapeDtypeStruct((B, 4, 56, 56), f32),
            jax.ShapeDtypeStruct((1,), f32),
        ],
        compiler_params=seq,
    )(h2, sc)
    qloss = lsum[0] * jnp.float32(1.2 / (B * 56 * 56 * 2))

    st3 = pl.pallas_call(
        _kd_body,
        grid=(G,),
        in_specs=[
            pl.BlockSpec((_C, 4, 56, 56), lambda i: (i, 0, 0, 0)),
            _smem(None), _smem(None),
        ],
        out_specs=pl.BlockSpec((2, 16), lambda i: (0, 0)),
        out_shape=jax.ShapeDtypeStruct((2, 16), f32),
        compiler_params=seq,
    )(z, p['dec_w1'], p['dec_b1'])

    mu3 = st3[0] / n1
    var3 = st3[1] / n1 - mu3 * mu3
    s3 = p['bn3_g'] / jnp.sqrt(var3 + 1e-5)
    t3 = (p['dec_b1'] - mu3) * s3 + p['bn3_b']

    out16 = pl.pallas_call(
        _ke_body,
        grid=(G,),
        in_specs=[
            pl.BlockSpec((_C, 4, 56, 56), lambda i: (i, 0, 0, 0)),
            _smem(None), _smem(None), _smem(None), _smem(None), _smem(None),
        ],
        out_specs=pl.BlockSpec((4, 4, _C, 56, 56), lambda i: (0, 0, i, 0, 0)),
        out_shape=jax.ShapeDtypeStruct((4, 4, B, 56, 56), f32),
        compiler_params=seq,
    )(z, p['dec_w1'], s3, t3, p['dec_w2'][:, 0], p['dec_b2'])

    output = out16.transpose(2, 3, 0, 4, 1).reshape(B, 1, 224, 224)
    aux = dict(st1=st1, h2=h2, st2=st2, z=z, lsum=lsum, st3=st3)
    return output, qloss, aux


def kernel(x, params):
    output, qloss, _ = _pipeline(x, params)
    return (output, qloss)
